# baseline (speedup 1.0000x reference)
import functools

import jax
import jax.numpy as jnp
from jax import lax
from jax.experimental import pallas as pl
from jax.experimental.pallas import tpu as pltpu
from jax.experimental.pallas import tpu_sc as plsc

VOCAB = 1000000
MAXLEN = 200
EMBED = 64
BATCH = 4096

NUM_CORES = 2
NUM_SUBCORES = 16
LANES = 16
NUM_WORKERS = NUM_CORES * NUM_SUBCORES

PAD = 128
POS_ROWS = MAXLEN * EMBED // PAD


def _sc_embed(x_t, t128, pos_lin):
    mesh = plsc.VectorSubcoreMesh(core_axis_name="c", subcore_axis_name="s")

    @functools.partial(
        pl.kernel,
        out_type=jax.ShapeDtypeStruct((MAXLEN, EMBED, BATCH), jnp.float32),
        mesh=mesh,
        compiler_params=pltpu.CompilerParams(needs_layout_passes=False),
        scratch_types=[
            pltpu.VMEM((PAD,), jnp.int32),
            pltpu.VMEM((PAD, PAD), jnp.float32),
            pltpu.VMEM((EMBED, PAD), jnp.float32),
            pltpu.VMEM((POS_ROWS, PAD), jnp.float32),
            pltpu.SemaphoreType.DMA,
        ],
    )
    def k(x_hbm, tok_hbm, pos_hbm, out_hbm, idx_v, g_v, o_v, pos_v, sem):
        wid = lax.axis_index("s") * NUM_CORES + lax.axis_index("c")
        b0 = wid * PAD
        pltpu.sync_copy(pos_hbm, pos_v)

        @pl.loop(0, MAXLEN)
        def _(s):
            pltpu.sync_copy(x_hbm.at[s, pl.ds(b0, PAD)], idx_v)
            pltpu.async_copy(tok_hbm.at[idx_v], g_v, sem).wait()

            prow = jnp.full((LANES,), s // 2, jnp.int32)
            pbase = (s % 2) * EMBED

            @pl.loop(0, EMBED)
            def _(e):
                plane = jnp.full((LANES,), pbase + e, jnp.int32)
                psv = plsc.load_gather(pos_v, [prow, plane])
                eidx = jnp.full((LANES,), e, jnp.int32)
                for c in range(PAD // LANES):
                    bidx = lax.iota(jnp.int32, LANES) + c * LANES
                    vals = plsc.load_gather(g_v, [bidx, eidx])
                    o_v[e, pl.ds(c * LANES, LANES)] = vals + psv

            pltpu.sync_copy(o_v, out_hbm.at[s, :, pl.ds(b0, PAD)])

    return k(x_t, t128, pos_lin)


def kernel(x, token_table, pos_table):
    x_t = jnp.transpose(x).astype(jnp.int32)
    t128 = jnp.pad(token_table, ((0, 0), (0, PAD - EMBED)))
    pos_lin = jnp.reshape(pos_table, (POS_ROWS, PAD))
    out3 = _sc_embed(x_t, t128, pos_lin)
    return jnp.transpose(out3, (2, 0, 1))


# speedup vs baseline: 1.8268x; 1.8268x over previous
import functools

import jax
import jax.numpy as jnp
from jax import lax
from jax.experimental import pallas as pl
from jax.experimental.pallas import tpu as pltpu
from jax.experimental.pallas import tpu_sc as plsc

VOCAB = 1000000
MAXLEN = 200
EMBED = 64
BATCH = 4096

NUM_CORES = 2
NUM_SUBCORES = 16
LANES = 16
NUM_WORKERS = NUM_CORES * NUM_SUBCORES

PAD = 128
SEQS_PER_W = BATCH // NUM_WORKERS


def _sc_embed(x, t128, pos128):
    mesh = plsc.VectorSubcoreMesh(core_axis_name="c", subcore_axis_name="s")

    @functools.partial(
        pl.kernel,
        out_type=jax.ShapeDtypeStruct((BATCH, MAXLEN, PAD), jnp.float32),
        mesh=mesh,
        compiler_params=pltpu.CompilerParams(needs_layout_passes=False),
        scratch_types=[
            pltpu.VMEM((MAXLEN,), jnp.int32),
            pltpu.VMEM((MAXLEN, PAD), jnp.float32),
            pltpu.VMEM((MAXLEN, PAD), jnp.float32),
            pltpu.SemaphoreType.DMA,
        ],
    )
    def k(x_hbm, tok_hbm, pos_hbm, out_hbm, idx_v, g_v, pos_v, sem):
        wid = lax.axis_index("s") * NUM_CORES + lax.axis_index("c")
        b0 = wid * SEQS_PER_W
        pltpu.sync_copy(pos_hbm, pos_v)

        @pl.loop(0, SEQS_PER_W)
        def _(j):
            b = b0 + j
            pltpu.sync_copy(x_hbm.at[b], idx_v)
            pltpu.async_copy(tok_hbm.at[idx_v], g_v, sem).wait()

            @pl.loop(0, MAXLEN)
            def _(r):
                for c in range(EMBED // LANES):
                    plsc.addupdate(
                        g_v.at[r, pl.ds(c * LANES, LANES)],
                        pos_v[r, pl.ds(c * LANES, LANES)],
                    )

            pltpu.sync_copy(g_v, out_hbm.at[b])

    return k(x, t128, pos128)


def kernel(x, token_table, pos_table):
    x32 = x.astype(jnp.int32)
    t128 = jnp.pad(token_table, ((0, 0), (0, PAD - EMBED)))
    pos128 = jnp.pad(pos_table, ((0, 0), (0, PAD - EMBED)))
    out = _sc_embed(x32, t128, pos128)
    return out[:, :, :EMBED]


# speedup vs baseline: 2.2492x; 1.2312x over previous
import functools

import jax
import jax.numpy as jnp
from jax import lax
from jax.experimental import pallas as pl
from jax.experimental.pallas import tpu as pltpu
from jax.experimental.pallas import tpu_sc as plsc

VOCAB = 1000000
MAXLEN = 200
EMBED = 64
BATCH = 4096

NUM_CORES = 2
NUM_SUBCORES = 16
LANES = 16
NUM_WORKERS = NUM_CORES * NUM_SUBCORES

PAD = 128
SEQS_PER_W = BATCH // NUM_WORKERS
NBUF = 2


def _sc_embed(x, t128, pos128):
    mesh = plsc.VectorSubcoreMesh(core_axis_name="c", subcore_axis_name="s")

    @functools.partial(
        pl.kernel,
        out_type=jax.ShapeDtypeStruct((BATCH, MAXLEN, PAD), jnp.float32),
        mesh=mesh,
        compiler_params=pltpu.CompilerParams(needs_layout_passes=False),
        scratch_types=[
            pltpu.VMEM((SEQS_PER_W * MAXLEN,), jnp.int32),
            pltpu.VMEM((NBUF, MAXLEN, PAD), jnp.float32),
            pltpu.VMEM((MAXLEN, PAD), jnp.float32),
            pltpu.SemaphoreType.DMA,
            pltpu.SemaphoreType.DMA,
            pltpu.SemaphoreType.DMA,
            pltpu.SemaphoreType.DMA,
        ],
    )
    def k(x_hbm, tok_hbm, pos_hbm, out_hbm, idx_all, g_v, pos_v,
          gsem0, gsem1, osem0, osem1):
        wid = lax.axis_index("s") * NUM_CORES + lax.axis_index("c")
        b0 = wid * SEQS_PER_W
        gsems = (gsem0, gsem1)
        osems = (osem0, osem1)

        pltpu.sync_copy(pos_hbm, pos_v)
        pltpu.sync_copy(x_hbm.at[pl.ds(b0 * MAXLEN, SEQS_PER_W * MAXLEN)], idx_all)

        def gather_start(j, slot):
            return pltpu.async_copy(
                tok_hbm.at[idx_all.at[pl.ds(j * MAXLEN, MAXLEN)]],
                g_v.at[slot], gsems[slot])

        def out_start(j, slot):
            return pltpu.async_copy(g_v.at[slot], out_hbm.at[b0 + j],
                                    osems[slot])

        def add_pos(slot):
            @pl.loop(0, MAXLEN, unroll=8)
            def _(r):
                for c in range(EMBED // LANES):
                    plsc.addupdate(
                        g_v.at[slot, r, pl.ds(c * LANES, LANES)],
                        pos_v[r, pl.ds(c * LANES, LANES)],
                    )

        GROUP = 16

        @pl.loop(0, SEQS_PER_W // GROUP)
        def _(gi):
            jb = gi * GROUP
            g = [None] * GROUP
            w = [None] * GROUP
            g[0] = gather_start(jb, 0)
            for t in range(GROUP):
                if t + 1 < GROUP:
                    if t >= 1:
                        w[t - 1].wait()
                    g[t + 1] = gather_start(jb + t + 1, (t + 1) % NBUF)
                g[t].wait()
                add_pos(t % NBUF)
                w[t] = out_start(jb + t, t % NBUF)
            w[GROUP - 2].wait()
            w[GROUP - 1].wait()

    return k(x, t128, pos128)


def kernel(x, token_table, pos_table):
    x32 = x.reshape(-1).astype(jnp.int32)
    t128 = jnp.pad(token_table, ((0, 0), (0, PAD - EMBED)))
    pos128 = jnp.pad(pos_table, ((0, 0), (0, PAD - EMBED)))
    out = _sc_embed(x32, t128, pos128)
    return out[:, :, :EMBED]
